# baseline (speedup 1.0000x reference)
import functools
import jax
import jax.numpy as jnp
from jax import lax
from jax.experimental import pallas as pl
from jax.experimental.pallas import tpu as pltpu
from jax.experimental.pallas import tpu_sc as plsc

B, S, D = 1024, 200, 128
NC, NS = 2, 16
NW = NC * NS
BW = B // NW
ROWS_W = BW * S
NB = 3
CH = 320
NCH = ROWS_W // CH


def _sc_body(in_hbm, pos_hbm, mask_hbm, out_hbm,
             buf0, buf1, buf2, pos_v, idx_v, mask_v, rows_v,
             is0, is1, is2, os0, os1, os2, sc_sem):
    wid = lax.axis_index("s") * NC + lax.axis_index("c")
    row_base = wid * ROWS_W
    bufs = (buf0, buf1, buf2)
    in_sems = (is0, is1, is2)
    out_sems = (os0, os1, os2)

    h_in = [None] * NCH
    h_out = [None] * NCH
    for g in range(NCH):
        if g >= NB:
            h_out[g - NB].wait()
        r0 = row_base + g * CH
        h_in[g] = pltpu.async_copy(in_hbm.at[pl.ds(r0, CH)], bufs[g % NB], in_sems[g % NB])
        if g >= 1:
            h_in[g - 1].wait()
            r0p = row_base + (g - 1) * CH
            h_out[g - 1] = pltpu.async_copy(bufs[(g - 1) % NB], out_hbm.at[pl.ds(r0p, CH)],
                                            out_sems[(g - 1) % NB])
    h_in[NCH - 1].wait()
    r0p = row_base + (NCH - 1) * CH
    h_out[NCH - 1] = pltpu.async_copy(bufs[(NCH - 1) % NB], out_hbm.at[pl.ds(r0p, CH)],
                                      out_sems[(NCH - 1) % NB])
    h_out[NCH - 3].wait()
    h_out[NCH - 2].wait()
    h_out[NCH - 1].wait()

    pltpu.sync_copy(pos_hbm.at[pl.ds(wid * BW, BW)], pos_v)
    for j in range(BW // 16):
        batch = wid * BW + j * 16 + lax.iota(jnp.int32, 16)
        idx_v[pl.ds(j * 16, 16)] = pos_v[pl.ds(j * 16, 16)] + batch * S
    pltpu.sync_copy(mask_hbm, mask_v)
    chunks = [mask_v[0, pl.ds(c * 16, 16)] for c in range(D // 16)]
    for r in range(BW):
        for c in range(D // 16):
            rows_v[r, pl.ds(c * 16, 16)] = chunks[c]
    pltpu.async_copy(rows_v, out_hbm.at[idx_v], sc_sem).wait()


_sc_call = functools.partial(
    pl.kernel,
    out_type=jax.ShapeDtypeStruct((B * S, D), jnp.float32),
    mesh=plsc.VectorSubcoreMesh(core_axis_name="c", subcore_axis_name="s"),
    scratch_types=[
        pltpu.VMEM((CH, D), jnp.float32),
        pltpu.VMEM((CH, D), jnp.float32),
        pltpu.VMEM((CH, D), jnp.float32),
        pltpu.VMEM((BW,), jnp.int32),
        pltpu.VMEM((BW,), jnp.int32),
        pltpu.VMEM((1, D), jnp.float32),
        pltpu.VMEM((BW, D), jnp.float32),
        pltpu.SemaphoreType.DMA,
        pltpu.SemaphoreType.DMA,
        pltpu.SemaphoreType.DMA,
        pltpu.SemaphoreType.DMA,
        pltpu.SemaphoreType.DMA,
        pltpu.SemaphoreType.DMA,
        pltpu.SemaphoreType.DMA,
    ],
)(_sc_body)


def kernel(inputs, categories, mask_positions, tokens_embedding):
    del categories
    pos = mask_positions.reshape(B).astype(jnp.int32)
    out = _sc_call(inputs.reshape(B * S, D), pos, tokens_embedding)
    return out.reshape(B, S, D)


# speedup vs baseline: 1.1466x; 1.1466x over previous
import functools
import jax
import jax.numpy as jnp
from jax import lax
from jax.experimental import pallas as pl
from jax.experimental.pallas import tpu as pltpu
from jax.experimental.pallas import tpu_sc as plsc

B, S, D = 1024, 200, 128
GB = 128
NC, NS = 2, 16
NW = NC * NS
BW = B // NW


def _copy_body(x_ref, o_ref):
    o_ref[...] = x_ref[...]


def _tc_copy(x):
    return pl.pallas_call(
        _copy_body,
        grid=(B // GB,),
        in_specs=[pl.BlockSpec((GB, S, D), lambda i: (i, 0, 0))],
        out_specs=pl.BlockSpec((GB, S, D), lambda i: (i, 0, 0)),
        out_shape=jax.ShapeDtypeStruct((B, S, D), jnp.float32),
        compiler_params=pltpu.CompilerParams(
            dimension_semantics=("arbitrary",),
        ),
    )(x)


def _sc_body(out_hbm, pos_hbm, mask_hbm, pos_v, idx_v, mask_v, rows_v, sem, sem2):
    wid = lax.axis_index("s") * NC + lax.axis_index("c")
    h_pos = pltpu.async_copy(pos_hbm.at[pl.ds(wid * BW, BW)], pos_v, sem)
    h_mask = pltpu.async_copy(mask_hbm, mask_v, sem2)
    h_pos.wait()
    for j in range(BW // 16):
        batch = wid * BW + j * 16 + lax.iota(jnp.int32, 16)
        idx_v[pl.ds(j * 16, 16)] = pos_v[pl.ds(j * 16, 16)] + batch * S
    h_mask.wait()
    chunks = [mask_v[0, pl.ds(c * 16, 16)] for c in range(D // 16)]
    for r in range(BW):
        for c in range(D // 16):
            rows_v[r, pl.ds(c * 16, 16)] = chunks[c]
    pltpu.async_copy(rows_v, out_hbm.at[idx_v], sem).wait()


_sc_scatter = functools.partial(
    pl.kernel,
    out_type=(),
    mesh=plsc.VectorSubcoreMesh(core_axis_name="c", subcore_axis_name="s"),
    scratch_types=[
        pltpu.VMEM((BW,), jnp.int32),
        pltpu.VMEM((BW,), jnp.int32),
        pltpu.VMEM((1, D), jnp.float32),
        pltpu.VMEM((BW, D), jnp.float32),
        pltpu.SemaphoreType.DMA,
        pltpu.SemaphoreType.DMA,
    ],
)(_sc_body)


def kernel(inputs, categories, mask_positions, tokens_embedding):
    del categories
    pos = mask_positions.reshape(B).astype(jnp.int32)
    copied = _tc_copy(inputs)
    out_ref = jax.new_ref(copied.reshape(B * S, D))
    _sc_scatter(out_ref, pos, tokens_embedding)
    return jax.freeze(out_ref).reshape(B, S, D)
